# baseline (speedup 1.0000x reference)
import functools

import jax
import jax.numpy as jnp
from jax import lax
from jax.experimental import pallas as pl
from jax.experimental.pallas import tpu as pltpu
from jax.experimental.pallas import tpu_sc as plsc

_S = 1000
_R = 1000
_B = 16384
_N = _S * _R
_NC = 1
_NS = 16
_NW = _NC * _NS
_BPW = _B // _NW
_JB = _BPW // 128
_L = 16

_sc_mesh = plsc.VectorSubcoreMesh(core_axis_name="c", subcore_axis_name="s",
                                  num_cores=_NC)


@functools.partial(
    pl.kernel,
    out_type=jax.ShapeDtypeStruct((_B,), jnp.float32),
    mesh=_sc_mesh,
    scratch_types=[
        pltpu.VMEM((_BPW,), jnp.int32),
        pltpu.VMEM((_BPW,), jnp.int32),
        pltpu.VMEM((_JB, 128), jnp.int32),
        pltpu.VMEM((_JB, 128), jnp.float32),
        pltpu.SemaphoreType.DMA,
        pltpu.SemaphoreType.DMA,
    ],
)
def _sc_gather(snd_hbm, rcv_hbm, wt_hbm, out_hbm,
               snd_v, rcv_v, idx_v, g_v, sem_io, sem_g):
    c = lax.axis_index("c")
    t = lax.axis_index("s")
    wid = t * _NC + c
    base = wid * _BPW
    tbl = wt_hbm.at[0]

    cp_s = pltpu.async_copy(snd_hbm.at[pl.ds(base, _BPW)], snd_v, sem_io)
    cp_r = pltpu.async_copy(rcv_hbm.at[pl.ds(base, _BPW)], rcv_v, sem_io)
    cp_s.wait()
    cp_r.wait()

    for j in range(_JB):
        for k in range(8):
            o = j * 128 + k * _L
            idx_v[j, pl.ds(k * _L, _L)] = snd_v[pl.ds(o, _L)] * _R + rcv_v[pl.ds(o, _L)]

    gathers = [
        pltpu.async_copy(tbl.at[idx_v.at[j]], g_v.at[j], sem_g)
        for j in range(_JB)
    ]
    for j, cp in enumerate(gathers):
        cp.wait()
        pltpu.sync_copy(g_v.at[j], out_hbm.at[pl.ds(base + j * 128, 128)])


def _stats_body(w_ref, m_ref, z_ref):
    x = w_ref[...]
    m = jnp.max(x)
    z = jnp.sum(jnp.exp(x - m))
    for j in range(_L):
        m_ref[j] = m
        z_ref[j] = z


_stats_tc = pl.pallas_call(
    _stats_body,
    out_shape=(
        jax.ShapeDtypeStruct((_L,), jnp.float32),
        jax.ShapeDtypeStruct((_L,), jnp.float32),
    ),
    in_specs=[pl.BlockSpec(memory_space=pltpu.VMEM)],
    out_specs=(
        pl.BlockSpec(memory_space=pltpu.SMEM),
        pl.BlockSpec(memory_space=pltpu.SMEM),
    ),
)


def _finalize_body(g_ref, m_ref, z_ref, o_ref):
    o_ref[...] = jnp.exp(g_ref[...] - m_ref[0]) * (1.0 / z_ref[0])


_finalize_tc = pl.pallas_call(
    _finalize_body,
    out_shape=jax.ShapeDtypeStruct((_B,), jnp.float32),
    in_specs=[
        pl.BlockSpec(memory_space=pltpu.VMEM),
        pl.BlockSpec(memory_space=pltpu.SMEM),
        pl.BlockSpec(memory_space=pltpu.SMEM),
    ],
    out_specs=pl.BlockSpec(memory_space=pltpu.VMEM),
)


def kernel(sender_idx, receiver_idx, commentary_weight):
    wt = commentary_weight.T
    g_raw = _sc_gather(sender_idx.astype(jnp.int32),
                       receiver_idx.astype(jnp.int32), wt)
    m16, z16 = _stats_tc(wt.reshape(8, _N // 8))
    return _finalize_tc(g_raw, m16, z16)


# speedup vs baseline: 2.7116x; 2.7116x over previous
import functools

import jax
import jax.numpy as jnp
from jax import lax
from jax.experimental import pallas as pl
from jax.experimental.pallas import tpu as pltpu
from jax.experimental.pallas import tpu_sc as plsc

_S = 1000
_R = 1000
_B = 16384
_N = _S * _R
_NC = 1
_NS = 16
_NW = _NC * _NS
_BPW = _B // _NW
_JB = _BPW // 128
_L = 16

_HALF = _N // 2
_CHUNK = 31248
_MAIN8 = 244
_REM = 1
_XTRA_OFF = _HALF + _CHUNK * _NS
_NEG_INF = float("-inf")


def _lane_bcast_reduce(v, op):
    lanes = lax.iota(jnp.int32, _L)
    for s in (8, 4, 2, 1):
        p = v.at[lanes ^ s].get(mode="promise_in_bounds")
        v = op(v, p)
    return v


_sc_mesh = plsc.VectorSubcoreMesh(core_axis_name="c", subcore_axis_name="s",
                                  num_cores=_NC)


@functools.partial(
    pl.kernel,
    out_type=(
        jax.ShapeDtypeStruct((_B,), jnp.float32),
        jax.ShapeDtypeStruct((_NS, _L), jnp.float32),
        jax.ShapeDtypeStruct((_NS, _L), jnp.float32),
    ),
    mesh=_sc_mesh,
    scratch_types=[
        pltpu.VMEM((_CHUNK,), jnp.float32),
        pltpu.VMEM((32,), jnp.float32),
        pltpu.VMEM((_BPW,), jnp.int32),
        pltpu.VMEM((_BPW,), jnp.int32),
        pltpu.VMEM((_JB, 128), jnp.int32),
        pltpu.VMEM((_JB, 128), jnp.float32),
        pltpu.VMEM((_L,), jnp.float32),
        pltpu.VMEM((_L,), jnp.float32),
        pltpu.SemaphoreType.DMA,
        pltpu.SemaphoreType.DMA,
        pltpu.SemaphoreType.DMA,
    ],
)
def _sc_half(snd_hbm, rcv_hbm, wt_hbm, g_hbm, mp_hbm, zp_hbm,
             chunk_v, xtra_v, snd_v, rcv_v, idx_v, g_v, m16_v, z16_v,
             sem_c, sem_io, sem_g):
    c = lax.axis_index("c")
    t = lax.axis_index("s")
    wid = t * _NC + c
    base = wid * _BPW
    tbl = wt_hbm.at[0]

    off = _HALF + t * _CHUNK
    cp_c = pltpu.async_copy(tbl.at[pl.ds(off, _CHUNK)], chunk_v, sem_c)
    cp_x = pltpu.async_copy(tbl.at[pl.ds(_XTRA_OFF, 32)], xtra_v, sem_io)
    cp_s = pltpu.async_copy(snd_hbm.at[pl.ds(base, _BPW)], snd_v, sem_io)
    cp_r = pltpu.async_copy(rcv_hbm.at[pl.ds(base, _BPW)], rcv_v, sem_io)
    cp_s.wait()
    cp_r.wait()
    cp_x.wait()

    for j in range(_JB):
        for k in range(8):
            o = j * 128 + k * _L
            idx_v[j, pl.ds(k * _L, _L)] = snd_v[pl.ds(o, _L)] * _R + rcv_v[pl.ds(o, _L)]
    gathers = [
        pltpu.async_copy(tbl.at[idx_v.at[j]], g_v.at[j], sem_g)
        for j in range(_JB)
    ]

    has_x = t < 2
    xv = xtra_v[pl.ds((t % 2) * _L, _L)]

    def body_max(i, mrun):
        b = i * 128
        for k in range(8):
            mrun = jnp.maximum(mrun, chunk_v[pl.ds(b + k * _L, _L)])
        return mrun

    cp_c.wait()
    mrun = lax.fori_loop(0, _MAIN8, body_max,
                         jnp.full((_L,), _NEG_INF, jnp.float32))
    for k in range(_REM):
        mrun = jnp.maximum(mrun, chunk_v[pl.ds(_MAIN8 * 128 + k * _L, _L)])
    mrun = jnp.where(has_x, jnp.maximum(mrun, xv), mrun)
    mvec = _lane_bcast_reduce(mrun, jnp.maximum)

    def body_sum(i, zrun):
        b = i * 128
        for k in range(8):
            zrun = zrun + jnp.exp(chunk_v[pl.ds(b + k * _L, _L)] - mvec)
        return zrun

    zrun = lax.fori_loop(0, _MAIN8, body_sum, jnp.zeros((_L,), jnp.float32))
    for k in range(_REM):
        zrun = zrun + jnp.exp(chunk_v[pl.ds(_MAIN8 * 128 + k * _L, _L)] - mvec)
    zrun = zrun + jnp.where(has_x, jnp.exp(xv - mvec),
                            jnp.zeros((_L,), jnp.float32))
    zvec = _lane_bcast_reduce(zrun, jnp.add)

    m16_v[...] = mvec
    z16_v[...] = zvec
    pltpu.sync_copy(m16_v, mp_hbm.at[t])
    pltpu.sync_copy(z16_v, zp_hbm.at[t])

    for j, cp in enumerate(gathers):
        cp.wait()
        pltpu.sync_copy(g_v.at[j], g_hbm.at[pl.ds(base + j * 128, 128)])


def _stats_body(w_ref, m_ref, z_ref):
    x = w_ref[...][:, :_HALF]
    m = jnp.max(x)
    z = jnp.sum(jnp.exp(x - m))
    for j in range(_L):
        m_ref[j] = m
        z_ref[j] = z


_stats_tc = pl.pallas_call(
    _stats_body,
    out_shape=(
        jax.ShapeDtypeStruct((_L,), jnp.float32),
        jax.ShapeDtypeStruct((_L,), jnp.float32),
    ),
    in_specs=[pl.BlockSpec(memory_space=pltpu.VMEM)],
    out_specs=(
        pl.BlockSpec(memory_space=pltpu.SMEM),
        pl.BlockSpec(memory_space=pltpu.SMEM),
    ),
)


def _finalize_body(g_ref, m_ref, z_ref, mp_ref, zp_ref, o_ref):
    m_tc = m_ref[0]
    z_tc = z_ref[0]
    mp = mp_ref[...][:, :1]
    zp = zp_ref[...][:, :1]
    gm = jnp.maximum(m_tc, jnp.max(mp))
    gz = z_tc * jnp.exp(m_tc - gm) + jnp.sum(zp * jnp.exp(mp - gm))
    o_ref[...] = jnp.exp(g_ref[...] - gm) * (1.0 / gz)


_finalize_tc = pl.pallas_call(
    _finalize_body,
    out_shape=jax.ShapeDtypeStruct((_B,), jnp.float32),
    in_specs=[
        pl.BlockSpec(memory_space=pltpu.VMEM),
        pl.BlockSpec(memory_space=pltpu.SMEM),
        pl.BlockSpec(memory_space=pltpu.SMEM),
        pl.BlockSpec(memory_space=pltpu.VMEM),
        pl.BlockSpec(memory_space=pltpu.VMEM),
    ],
    out_specs=pl.BlockSpec(memory_space=pltpu.VMEM),
)


def kernel(sender_idx, receiver_idx, commentary_weight):
    wt = commentary_weight.T
    g_raw, mp, zp = _sc_half(sender_idx.astype(jnp.int32),
                             receiver_idx.astype(jnp.int32), wt)
    m16, z16 = _stats_tc(wt)
    return _finalize_tc(g_raw, m16, z16, mp, zp)


# speedup vs baseline: 2.8180x; 1.0392x over previous
import functools

import jax
import jax.numpy as jnp
from jax import lax
from jax.experimental import pallas as pl
from jax.experimental.pallas import tpu as pltpu
from jax.experimental.pallas import tpu_sc as plsc

_S = 1000
_R = 1000
_B = 16384
_N = _S * _R
_NC = 1
_NS = 16
_NW = _NC * _NS
_BPW = _B // _NW
_JB = _BPW // 128
_L = 16

_HALF = 630000
_CHUNK = 23120
_MAIN8 = 180
_REM = 5
_XTRA_OFF = _HALF + _CHUNK * _NS
_NEG_INF = float("-inf")


def _lane_bcast_reduce(v, op):
    lanes = lax.iota(jnp.int32, _L)
    for s in (8, 4, 2, 1):
        p = v.at[lanes ^ s].get(mode="promise_in_bounds")
        v = op(v, p)
    return v


_sc_mesh = plsc.VectorSubcoreMesh(core_axis_name="c", subcore_axis_name="s",
                                  num_cores=_NC)


@functools.partial(
    pl.kernel,
    out_type=(
        jax.ShapeDtypeStruct((_B,), jnp.float32),
        jax.ShapeDtypeStruct((_NS, _L), jnp.float32),
        jax.ShapeDtypeStruct((_NS, _L), jnp.float32),
    ),
    mesh=_sc_mesh,
    scratch_types=[
        pltpu.VMEM((_CHUNK,), jnp.float32),
        pltpu.VMEM((80,), jnp.float32),
        pltpu.VMEM((_BPW,), jnp.int32),
        pltpu.VMEM((_BPW,), jnp.int32),
        pltpu.VMEM((_JB, 128), jnp.int32),
        pltpu.VMEM((_JB, 128), jnp.float32),
        pltpu.VMEM((_L,), jnp.float32),
        pltpu.VMEM((_L,), jnp.float32),
        pltpu.SemaphoreType.DMA,
        pltpu.SemaphoreType.DMA,
        pltpu.SemaphoreType.DMA,
    ],
)
def _sc_half(snd_hbm, rcv_hbm, wt_hbm, g_hbm, mp_hbm, zp_hbm,
             chunk_v, xtra_v, snd_v, rcv_v, idx_v, g_v, m16_v, z16_v,
             sem_c, sem_io, sem_g):
    c = lax.axis_index("c")
    t = lax.axis_index("s")
    wid = t * _NC + c
    base = wid * _BPW
    tbl = wt_hbm.at[0]

    off = _HALF + t * _CHUNK
    cp_c = pltpu.async_copy(tbl.at[pl.ds(off, _CHUNK)], chunk_v, sem_c)
    cp_x = pltpu.async_copy(tbl.at[pl.ds(_XTRA_OFF, 80)], xtra_v, sem_io)
    cp_s = pltpu.async_copy(snd_hbm.at[pl.ds(base, _BPW)], snd_v, sem_io)
    cp_r = pltpu.async_copy(rcv_hbm.at[pl.ds(base, _BPW)], rcv_v, sem_io)
    cp_s.wait()
    cp_r.wait()
    cp_x.wait()

    for j in range(_JB):
        for k in range(8):
            o = j * 128 + k * _L
            idx_v[j, pl.ds(k * _L, _L)] = snd_v[pl.ds(o, _L)] * _R + rcv_v[pl.ds(o, _L)]
    gathers = [
        pltpu.async_copy(tbl.at[idx_v.at[j]], g_v.at[j], sem_g)
        for j in range(_JB)
    ]

    has_x = t < 5
    xv = xtra_v[pl.ds((t % 5) * _L, _L)]

    def body_max(i, mrun):
        b = i * 128
        for k in range(8):
            mrun = jnp.maximum(mrun, chunk_v[pl.ds(b + k * _L, _L)])
        return mrun

    cp_c.wait()
    mrun = lax.fori_loop(0, _MAIN8, body_max,
                         jnp.full((_L,), _NEG_INF, jnp.float32))
    for k in range(_REM):
        mrun = jnp.maximum(mrun, chunk_v[pl.ds(_MAIN8 * 128 + k * _L, _L)])
    mrun = jnp.where(has_x, jnp.maximum(mrun, xv), mrun)
    mvec = _lane_bcast_reduce(mrun, jnp.maximum)

    def body_sum(i, zrun):
        b = i * 128
        for k in range(8):
            zrun = zrun + jnp.exp(chunk_v[pl.ds(b + k * _L, _L)] - mvec)
        return zrun

    zrun = lax.fori_loop(0, _MAIN8, body_sum, jnp.zeros((_L,), jnp.float32))
    for k in range(_REM):
        zrun = zrun + jnp.exp(chunk_v[pl.ds(_MAIN8 * 128 + k * _L, _L)] - mvec)
    zrun = zrun + jnp.where(has_x, jnp.exp(xv - mvec),
                            jnp.zeros((_L,), jnp.float32))
    zvec = _lane_bcast_reduce(zrun, jnp.add)

    m16_v[...] = mvec
    z16_v[...] = zvec
    pltpu.sync_copy(m16_v, mp_hbm.at[t])
    pltpu.sync_copy(z16_v, zp_hbm.at[t])

    for j, cp in enumerate(gathers):
        cp.wait()
        pltpu.sync_copy(g_v.at[j], g_hbm.at[pl.ds(base + j * 128, 128)])


def _stats_body(w_ref, m_ref, z_ref):
    x = w_ref[...][:, :_HALF]
    m = jnp.max(x)
    z = jnp.sum(jnp.exp(x - m))
    for j in range(_L):
        m_ref[j] = m
        z_ref[j] = z


_stats_tc = pl.pallas_call(
    _stats_body,
    out_shape=(
        jax.ShapeDtypeStruct((_L,), jnp.float32),
        jax.ShapeDtypeStruct((_L,), jnp.float32),
    ),
    in_specs=[pl.BlockSpec(memory_space=pltpu.VMEM)],
    out_specs=(
        pl.BlockSpec(memory_space=pltpu.SMEM),
        pl.BlockSpec(memory_space=pltpu.SMEM),
    ),
)


def _finalize_body(g_ref, m_ref, z_ref, mp_ref, zp_ref, o_ref):
    m_tc = m_ref[0]
    z_tc = z_ref[0]
    mp = mp_ref[...][:, :1]
    zp = zp_ref[...][:, :1]
    gm = jnp.maximum(m_tc, jnp.max(mp))
    gz = z_tc * jnp.exp(m_tc - gm) + jnp.sum(zp * jnp.exp(mp - gm))
    o_ref[...] = jnp.exp(g_ref[...] - gm) * (1.0 / gz)


_finalize_tc = pl.pallas_call(
    _finalize_body,
    out_shape=jax.ShapeDtypeStruct((_B,), jnp.float32),
    in_specs=[
        pl.BlockSpec(memory_space=pltpu.VMEM),
        pl.BlockSpec(memory_space=pltpu.SMEM),
        pl.BlockSpec(memory_space=pltpu.SMEM),
        pl.BlockSpec(memory_space=pltpu.VMEM),
        pl.BlockSpec(memory_space=pltpu.VMEM),
    ],
    out_specs=pl.BlockSpec(memory_space=pltpu.VMEM),
)


def kernel(sender_idx, receiver_idx, commentary_weight):
    wt = commentary_weight.T
    g_raw, mp, zp = _sc_half(sender_idx.astype(jnp.int32),
                             receiver_idx.astype(jnp.int32), wt)
    m16, z16 = _stats_tc(wt)
    return _finalize_tc(g_raw, m16, z16, mp, zp)
